# baseline (speedup 1.0000x reference)
import functools

import jax
import jax.numpy as jnp
from jax import lax
from jax.experimental import pallas as pl
from jax.experimental.pallas import tpu as pltpu
from jax.experimental.pallas import tpu_sc as plsc

VOCAB = 1000000
EMBED_DIM = 64
BATCH = 16384
TEXT_DIM = 64
HIDDEN = 128
ROW2 = 2 * EMBED_DIM

NC = 2
NS = 16
NW = NC * NS
B_PER_W = BATCH // NW
CHUNK = 128
NCHUNK = B_PER_W // CHUNK

BQ = 4096
GRID_T = 64
H = BQ * GRID_T
LAST_BLK = (VOCAB - 1) // BQ

BM = 4096
GRID_M = BATCH // BM


def _pack2(lo_bf, hi_bf):
    lo = lax.bitcast_convert_type(lo_bf, jnp.uint16)
    hi = lax.bitcast_convert_type(hi_bf, jnp.uint16)
    return (hi.astype(jnp.uint32) << 16) | lo.astype(jnp.uint32)


def _untranspose_body(t1_ref, t2_ref, t3_ref, t4_ref, out_ref):
    eye = (
        lax.broadcasted_iota(jnp.int32, (EMBED_DIM, EMBED_DIM), 0)
        == lax.broadcasted_iota(jnp.int32, (EMBED_DIM, EMBED_DIM), 1)
    ).astype(jnp.bfloat16)
    dn = (((0,), (0,)), ((), ()))
    tA = lax.dot_general(t1_ref[...].astype(jnp.bfloat16), eye, dn,
                         preferred_element_type=jnp.float32).astype(jnp.bfloat16)
    tB = lax.dot_general(t2_ref[...].astype(jnp.bfloat16), eye, dn,
                         preferred_element_type=jnp.float32).astype(jnp.bfloat16)
    tC = lax.dot_general(t3_ref[...].astype(jnp.bfloat16), eye, dn,
                         preferred_element_type=jnp.float32).astype(jnp.bfloat16)
    tD = lax.dot_general(t4_ref[...].astype(jnp.bfloat16), eye, dn,
                         preferred_element_type=jnp.float32).astype(jnp.bfloat16)
    out_ref[...] = jnp.concatenate([_pack2(tA, tB), _pack2(tC, tD)], axis=1)


@jax.jit
def _tc_untranspose(tT):
    def view(k):
        return pl.BlockSpec(
            (EMBED_DIM, BQ),
            lambda i, k=k: (0, jnp.minimum(i + k * GRID_T, LAST_BLK)),
        )

    return pl.pallas_call(
        _untranspose_body,
        out_shape=jax.ShapeDtypeStruct((H, ROW2), jnp.uint32),
        grid=(GRID_T,),
        in_specs=[view(0), view(1), view(2), view(3)],
        out_specs=pl.BlockSpec((BQ, ROW2), lambda i: (i, 0)),
        compiler_params=pltpu.CompilerParams(fuse_transposed_lhs_in_matmul=True),
    )(tT, tT, tT, tT)


def _gather_body(table_hbm, idx_hbm, out_hbm, idx_v, rows_v, sem):
    wid = lax.axis_index("s") * NC + lax.axis_index("c")
    base = wid * B_PER_W
    pltpu.sync_copy(idx_hbm.at[wid], idx_v)
    copies = []
    for j in range(NCHUNK):
        copies.append(
            pltpu.async_copy(
                table_hbm.at[idx_v.at[j]],
                rows_v.at[pl.ds(j * CHUNK, CHUNK)],
                sem,
            )
        )
    for c in copies:
        c.wait()
    pltpu.sync_copy(rows_v, out_hbm.at[pl.ds(base, B_PER_W)])


@jax.jit
def _sc_gather(table4, idx):
    mesh = plsc.VectorSubcoreMesh(core_axis_name="c", subcore_axis_name="s")
    return pl.kernel(
        _gather_body,
        out_type=jax.ShapeDtypeStruct((BATCH, ROW2), jnp.uint32),
        mesh=mesh,
        scratch_types=[
            pltpu.VMEM((NCHUNK, CHUNK), jnp.int32),
            pltpu.VMEM((B_PER_W, ROW2), jnp.uint32),
            pltpu.SemaphoreType.DMA,
        ],
        compiler_params=pltpu.CompilerParams(use_tc_tiling_on_sc=True),
    )(table4, idx)


def _mlp_body(g_ref, selhi_ref, selgrp_ref, at_ref, ht_ref,
              w1a_ref, w1b_ref, w1c_ref, b1_ref, w2_ref, b2_ref, out_ref):
    u = g_ref[...]
    lo = lax.bitcast_convert_type((u & jnp.uint32(0xFFFF)).astype(jnp.uint16),
                                  jnp.bfloat16)
    hi = lax.bitcast_convert_type((u >> 16).astype(jnp.uint16), jnp.bfloat16)
    sel = jnp.where(selhi_ref[...] > 0.5, hi, lo).astype(jnp.float32)
    g = jnp.where(selgrp_ref[...] > 0.5,
                  sel[:, EMBED_DIM:], sel[:, :EMBED_DIM])
    dn_t = (((0,), (1,)), ((), ()))
    dn_n = (((0,), (0,)), ((), ()))
    x = (
        lax.dot_general(w1a_ref[...], g, dn_t, preferred_element_type=jnp.float32)
        + lax.dot_general(w1b_ref[...], at_ref[...], dn_n, preferred_element_type=jnp.float32)
        + lax.dot_general(w1c_ref[...], ht_ref[...], dn_n, preferred_element_type=jnp.float32)
        + b1_ref[...]
    )
    x = jnp.maximum(x, 0.0)
    y = lax.dot_general(w2_ref[...], x, dn_n, preferred_element_type=jnp.float32) + b2_ref[...]
    out_ref[...] = jnp.maximum(y, 0.0)


@jax.jit
def _tc_mlp(gathered4, selhi, selgrp, aboutT, headT, w1a, w1b, w1c, b1, w2, b2):
    blk = lambda i: (i, 0)
    blkT = lambda i: (0, i)
    rep = lambda i: (0, 0)
    return pl.pallas_call(
        _mlp_body,
        out_shape=jax.ShapeDtypeStruct((EMBED_DIM, BATCH), jnp.float32),
        grid=(GRID_M,),
        in_specs=[
            pl.BlockSpec((BM, ROW2), blk),
            pl.BlockSpec((BM, 1), blk),
            pl.BlockSpec((BM, 1), blk),
            pl.BlockSpec((TEXT_DIM, BM), blkT),
            pl.BlockSpec((TEXT_DIM, BM), blkT),
            pl.BlockSpec((EMBED_DIM, HIDDEN), rep),
            pl.BlockSpec((TEXT_DIM, HIDDEN), rep),
            pl.BlockSpec((TEXT_DIM, HIDDEN), rep),
            pl.BlockSpec((HIDDEN, 1), rep),
            pl.BlockSpec((HIDDEN, EMBED_DIM), rep),
            pl.BlockSpec((EMBED_DIM, 1), rep),
        ],
        out_specs=pl.BlockSpec((EMBED_DIM, BM), blkT),
    )(gathered4, selhi, selgrp, aboutT, headT, w1a, w1b, w1c, b1, w2, b2)


def kernel(user_id, about_embedding, headline_embedding, emb_table, W1, b1, W2, b2):
    uid = user_id.astype(jnp.int32)
    quad = uid >> 18
    idx = (uid & (H - 1)).reshape(NW, NCHUNK, CHUNK)
    selhi = (quad & 1).astype(jnp.float32).reshape(BATCH, 1)
    selgrp = (quad >> 1).astype(jnp.float32).reshape(BATCH, 1)
    table4 = _tc_untranspose(emb_table.T)
    gathered4 = _sc_gather(table4, idx)
    w1a = W1[:EMBED_DIM]
    w1b = W1[EMBED_DIM:EMBED_DIM + TEXT_DIM]
    w1c = W1[EMBED_DIM + TEXT_DIM:]
    yt = _tc_mlp(
        gathered4, selhi, selgrp, about_embedding.T, headline_embedding.T,
        w1a, w1b, w1c, b1.reshape(HIDDEN, 1), W2, b2.reshape(EMBED_DIM, 1),
    )
    return yt.T


# speedup vs baseline: 1.1026x; 1.1026x over previous
import functools

import jax
import jax.numpy as jnp
from jax import lax
from jax.experimental import pallas as pl
from jax.experimental.pallas import tpu as pltpu
from jax.experimental.pallas import tpu_sc as plsc

VOCAB = 1000000
EMBED_DIM = 64
BATCH = 16384
TEXT_DIM = 64
HIDDEN = 128
ROW2 = 2 * EMBED_DIM

NC = 2
NS = 16
NW = NC * NS
B_PER_W = BATCH // NW
CHUNK = 128
NCHUNK = B_PER_W // CHUNK

BQ = 8192
GRID_T = 32
H = BQ * GRID_T
LAST_BLK = (VOCAB - 1) // BQ

BM = 4096
GRID_M = BATCH // BM


def _pack2(lo_bf, hi_bf):
    lo = lax.bitcast_convert_type(lo_bf, jnp.uint16)
    hi = lax.bitcast_convert_type(hi_bf, jnp.uint16)
    return (hi.astype(jnp.uint32) << 16) | lo.astype(jnp.uint32)


def _untranspose_body(t1_ref, t2_ref, t3_ref, t4_ref, out_ref):
    eye = (
        lax.broadcasted_iota(jnp.int32, (EMBED_DIM, EMBED_DIM), 0)
        == lax.broadcasted_iota(jnp.int32, (EMBED_DIM, EMBED_DIM), 1)
    ).astype(jnp.bfloat16)
    dn = (((0,), (0,)), ((), ()))
    tA = lax.dot_general(t1_ref[...].astype(jnp.bfloat16), eye, dn,
                         preferred_element_type=jnp.float32).astype(jnp.bfloat16)
    tB = lax.dot_general(t2_ref[...].astype(jnp.bfloat16), eye, dn,
                         preferred_element_type=jnp.float32).astype(jnp.bfloat16)
    tC = lax.dot_general(t3_ref[...].astype(jnp.bfloat16), eye, dn,
                         preferred_element_type=jnp.float32).astype(jnp.bfloat16)
    tD = lax.dot_general(t4_ref[...].astype(jnp.bfloat16), eye, dn,
                         preferred_element_type=jnp.float32).astype(jnp.bfloat16)
    out_ref[...] = jnp.concatenate([_pack2(tA, tB), _pack2(tC, tD)], axis=1)


@jax.jit
def _tc_untranspose(tT):
    def view(k):
        return pl.BlockSpec(
            (EMBED_DIM, BQ),
            lambda i, k=k: (0, jnp.minimum(i + k * GRID_T, LAST_BLK)),
        )

    return pl.pallas_call(
        _untranspose_body,
        out_shape=jax.ShapeDtypeStruct((H, ROW2), jnp.uint32),
        grid=(GRID_T,),
        in_specs=[view(0), view(1), view(2), view(3)],
        out_specs=pl.BlockSpec((BQ, ROW2), lambda i: (i, 0)),
        compiler_params=pltpu.CompilerParams(fuse_transposed_lhs_in_matmul=True),
    )(tT, tT, tT, tT)


def _gather_body(table_hbm, idx_hbm, out_hbm, idx_v, rows_v, sem):
    wid = lax.axis_index("s") * NC + lax.axis_index("c")
    base = wid * B_PER_W
    pltpu.sync_copy(idx_hbm.at[wid], idx_v)
    copies = []
    for j in range(NCHUNK):
        copies.append(
            pltpu.async_copy(
                table_hbm.at[idx_v.at[j]],
                rows_v.at[pl.ds(j * CHUNK, CHUNK)],
                sem,
            )
        )
    for c in copies:
        c.wait()
    pltpu.sync_copy(rows_v, out_hbm.at[pl.ds(base, B_PER_W)])


@jax.jit
def _sc_gather(table4, idx):
    mesh = plsc.VectorSubcoreMesh(core_axis_name="c", subcore_axis_name="s")
    return pl.kernel(
        _gather_body,
        out_type=jax.ShapeDtypeStruct((BATCH, ROW2), jnp.uint32),
        mesh=mesh,
        scratch_types=[
            pltpu.VMEM((NCHUNK, CHUNK), jnp.int32),
            pltpu.VMEM((B_PER_W, ROW2), jnp.uint32),
            pltpu.SemaphoreType.DMA,
        ],
        compiler_params=pltpu.CompilerParams(use_tc_tiling_on_sc=True),
    )(table4, idx)


def _mlp_body(g_ref, selhi_ref, selgrp_ref, at_ref, ht_ref,
              w1a_ref, w1b_ref, w1c_ref, b1_ref, w2_ref, b2_ref, out_ref):
    u = g_ref[...]
    lo = lax.bitcast_convert_type((u & jnp.uint32(0xFFFF)).astype(jnp.uint16),
                                  jnp.bfloat16)
    hi = lax.bitcast_convert_type((u >> 16).astype(jnp.uint16), jnp.bfloat16)
    sel = jnp.where(selhi_ref[...] > 0.5, hi, lo).astype(jnp.float32)
    g = jnp.where(selgrp_ref[...] > 0.5,
                  sel[:, EMBED_DIM:], sel[:, :EMBED_DIM])
    dn_t = (((0,), (1,)), ((), ()))
    dn_n = (((0,), (0,)), ((), ()))
    x = (
        lax.dot_general(w1a_ref[...], g, dn_t, preferred_element_type=jnp.float32)
        + lax.dot_general(w1b_ref[...], at_ref[...], dn_n, preferred_element_type=jnp.float32)
        + lax.dot_general(w1c_ref[...], ht_ref[...], dn_n, preferred_element_type=jnp.float32)
        + b1_ref[...]
    )
    x = jnp.maximum(x, 0.0)
    y = lax.dot_general(w2_ref[...], x, dn_n, preferred_element_type=jnp.float32) + b2_ref[...]
    out_ref[...] = jnp.maximum(y, 0.0)


@jax.jit
def _tc_mlp(gathered4, selhi, selgrp, aboutT, headT, w1a, w1b, w1c, b1, w2, b2):
    blk = lambda i: (i, 0)
    blkT = lambda i: (0, i)
    rep = lambda i: (0, 0)
    return pl.pallas_call(
        _mlp_body,
        out_shape=jax.ShapeDtypeStruct((EMBED_DIM, BATCH), jnp.float32),
        grid=(GRID_M,),
        in_specs=[
            pl.BlockSpec((BM, ROW2), blk),
            pl.BlockSpec((BM, 1), blk),
            pl.BlockSpec((BM, 1), blk),
            pl.BlockSpec((TEXT_DIM, BM), blkT),
            pl.BlockSpec((TEXT_DIM, BM), blkT),
            pl.BlockSpec((EMBED_DIM, HIDDEN), rep),
            pl.BlockSpec((TEXT_DIM, HIDDEN), rep),
            pl.BlockSpec((TEXT_DIM, HIDDEN), rep),
            pl.BlockSpec((HIDDEN, 1), rep),
            pl.BlockSpec((HIDDEN, EMBED_DIM), rep),
            pl.BlockSpec((EMBED_DIM, 1), rep),
        ],
        out_specs=pl.BlockSpec((EMBED_DIM, BM), blkT),
    )(gathered4, selhi, selgrp, aboutT, headT, w1a, w1b, w1c, b1, w2, b2)


def kernel(user_id, about_embedding, headline_embedding, emb_table, W1, b1, W2, b2):
    uid = user_id.astype(jnp.int32)
    quad = uid >> 18
    idx = (uid & (H - 1)).reshape(NW, NCHUNK, CHUNK)
    selhi = (quad & 1).astype(jnp.float32).reshape(BATCH, 1)
    selgrp = (quad >> 1).astype(jnp.float32).reshape(BATCH, 1)
    table4 = _tc_untranspose(emb_table.T)
    gathered4 = _sc_gather(table4, idx)
    w1a = W1[:EMBED_DIM]
    w1b = W1[EMBED_DIM:EMBED_DIM + TEXT_DIM]
    w1c = W1[EMBED_DIM + TEXT_DIM:]
    yt = _tc_mlp(
        gathered4, selhi, selgrp, about_embedding.T, headline_embedding.T,
        w1a, w1b, w1c, b1.reshape(HIDDEN, 1), W2, b2.reshape(EMBED_DIM, 1),
    )
    return yt.T
